# initial kernel scaffold (unmeasured)
import jax
import jax.numpy as jnp
from jax import lax
from jax.experimental import pallas as pl
from jax.experimental.pallas import tpu as pltpu

N_DEV = 16
B, SQ, D = 4, 256, 1024
H_LOC = 8
G_SIZE = 4
HKV_LOC = 2
DH = 128
ROWS = B * SQ
CHUNK = ROWS // N_DEV
SCALE = 0.08838834764831843


def kernel(x, Wq, Wo, K_ext, V_ext):
    my = lax.axis_index("i")
    x2 = x.reshape(ROWS, D).astype(jnp.bfloat16)
    Wq_b = Wq.astype(jnp.bfloat16)
    Wo_b = Wo.astype(jnp.bfloat16)
    K_loc = lax.dynamic_slice_in_dim(K_ext, my * HKV_LOC, HKV_LOC, axis=2)
    V_loc = lax.dynamic_slice_in_dim(V_ext, my * HKV_LOC, HKV_LOC, axis=2)
    K_loc = K_loc.astype(jnp.bfloat16)
    V_loc = V_loc.astype(jnp.bfloat16)

    def body(x_ref, wq_ref, wo_ref, k_ref, v_ref, out_ref,
             attn_ref, part_ref, recv_ref,
             send_a, recv_a, send_b, recv_b):
        my_i = lax.axis_index("i")

        barrier_sem = pltpu.get_barrier_semaphore()
        for t in range(1, N_DEV):
            pl.semaphore_signal(
                barrier_sem, inc=1,
                device_id=((my_i + t) % N_DEV,),
                device_id_type=pl.DeviceIdType.MESH,
            )
        pl.semaphore_wait(barrier_sem, N_DEV - 1)

        q_all = lax.dot_general(
            x_ref[...], wq_ref[...], (((1,), (0,)), ((), ())),
            preferred_element_type=jnp.float32,
        ).astype(jnp.bfloat16)

        for b in range(B):
            for h in range(H_LOC):
                g = h // G_SIZE
                q = q_all[b * SQ:(b + 1) * SQ, h * DH:(h + 1) * DH]
                k = k_ref[b, :, g, :]
                v = v_ref[b, :, g, :]
                s = lax.dot_general(
                    q, k, (((1,), (1,)), ((), ())),
                    preferred_element_type=jnp.float32,
                ) * SCALE
                m = jnp.max(s, axis=1, keepdims=True)
                p = jnp.exp(s - m)
                l = jnp.sum(p, axis=1, keepdims=True)
                o = lax.dot_general(
                    p.astype(jnp.bfloat16), v, (((1,), (0,)), ((), ())),
                    preferred_element_type=jnp.float32,
                )
                attn_ref[b * SQ:(b + 1) * SQ, h * DH:(h + 1) * DH] = (
                    o / l
                ).astype(jnp.bfloat16)

        part_ref[...] = lax.dot_general(
            attn_ref[...], wo_ref[...], (((1,), (0,)), ((), ())),
            preferred_element_type=jnp.float32,
        )

        for t in range(1, N_DEV):
            tgt = (my_i + t) % N_DEV
            pltpu.make_async_remote_copy(
                src_ref=part_ref.at[pl.ds(tgt * CHUNK, CHUNK), :],
                dst_ref=recv_ref.at[t],
                send_sem=send_a.at[t],
                recv_sem=recv_a.at[t],
                device_id=(tgt,),
                device_id_type=pl.DeviceIdType.MESH,
            ).start()

        acc = part_ref[pl.ds(my_i * CHUNK, CHUNK), :]
        for t in range(1, N_DEV):
            pltpu.make_async_remote_copy(
                src_ref=part_ref.at[pl.ds(0, CHUNK), :],
                dst_ref=recv_ref.at[t],
                send_sem=send_a.at[t],
                recv_sem=recv_a.at[t],
                device_id=(0,),
                device_id_type=pl.DeviceIdType.MESH,
            ).wait_recv()
            acc = acc + recv_ref[t]

        out_ref[pl.ds(my_i * CHUNK, CHUNK), :] = acc

        for t in range(1, N_DEV):
            tgt = (my_i + t) % N_DEV
            pltpu.make_async_remote_copy(
                src_ref=out_ref.at[pl.ds(my_i * CHUNK, CHUNK), :],
                dst_ref=out_ref.at[pl.ds(my_i * CHUNK, CHUNK), :],
                send_sem=send_b.at[t],
                recv_sem=recv_b.at[t],
                device_id=(tgt,),
                device_id_type=pl.DeviceIdType.MESH,
            ).start()

        for t in range(1, N_DEV):
            src_id = (my_i - t) % N_DEV
            pltpu.make_async_remote_copy(
                src_ref=out_ref.at[pl.ds(src_id * CHUNK, CHUNK), :],
                dst_ref=out_ref.at[pl.ds(src_id * CHUNK, CHUNK), :],
                send_sem=send_b.at[t],
                recv_sem=recv_b.at[t],
                device_id=(src_id,),
                device_id_type=pl.DeviceIdType.MESH,
            ).wait_recv()

        for t in range(1, N_DEV):
            pltpu.make_async_remote_copy(
                src_ref=part_ref.at[pl.ds(0, CHUNK), :],
                dst_ref=recv_ref.at[t],
                send_sem=send_a.at[t],
                recv_sem=recv_a.at[t],
                device_id=(0,),
                device_id_type=pl.DeviceIdType.MESH,
            ).wait_send()
            pltpu.make_async_remote_copy(
                src_ref=out_ref.at[pl.ds(0, CHUNK), :],
                dst_ref=out_ref.at[pl.ds(0, CHUNK), :],
                send_sem=send_b.at[t],
                recv_sem=recv_b.at[t],
                device_id=(0,),
                device_id_type=pl.DeviceIdType.MESH,
            ).wait_send()

    out = pl.pallas_call(
        body,
        out_shape=jax.ShapeDtypeStruct((ROWS, D), jnp.float32),
        in_specs=[pl.BlockSpec(memory_space=pltpu.VMEM)] * 5,
        out_specs=pl.BlockSpec(memory_space=pltpu.VMEM),
        scratch_shapes=[
            pltpu.VMEM((ROWS, H_LOC * DH), jnp.bfloat16),
            pltpu.VMEM((ROWS, D), jnp.float32),
            pltpu.VMEM((N_DEV, CHUNK, D), jnp.float32),
            pltpu.SemaphoreType.DMA((N_DEV,)),
            pltpu.SemaphoreType.DMA((N_DEV,)),
            pltpu.SemaphoreType.DMA((N_DEV,)),
            pltpu.SemaphoreType.DMA((N_DEV,)),
        ],
        compiler_params=pltpu.CompilerParams(collective_id=0),
    )(x2, Wq_b, Wo_b, K_loc, V_loc)
    return out.reshape(B, SQ, D)


# baseline (device time: 205017 ns/iter reference)
import os

import jax
import jax.numpy as jnp
from jax import lax
from jax.experimental import pallas as pl
from jax.experimental.pallas import tpu as pltpu

_STAGE = int(os.environ.get("KERNEL_STAGE", "3"))

N_DEV = 16
B, SQ, D = 4, 256, 1024
H_LOC = 8
G_SIZE = 4
HKV_LOC = 2
DH = 128
ROWS = B * SQ
CHUNK = ROWS // N_DEV
SCALE = 0.08838834764831843


def kernel(x, Wq, Wo, K_ext, V_ext):
    my = lax.axis_index("i")
    x2 = x.reshape(ROWS, D).astype(jnp.bfloat16)
    Wq_b = Wq.astype(jnp.bfloat16)
    Wo_b = Wo.astype(jnp.bfloat16)
    K_loc = lax.dynamic_slice_in_dim(K_ext, my * HKV_LOC, HKV_LOC, axis=2)
    V_loc = lax.dynamic_slice_in_dim(V_ext, my * HKV_LOC, HKV_LOC, axis=2)
    K_loc = K_loc.astype(jnp.bfloat16)
    V_loc = V_loc.astype(jnp.bfloat16)

    def body(x_ref, wq_ref, wo_ref, k_ref, v_ref, out_ref,
             attn_ref, part_ref, recv_ref,
             send_a, recv_a, send_b, recv_b):
        my_i = lax.axis_index("i")

        if _STAGE >= 1:
            barrier_sem = pltpu.get_barrier_semaphore()
            for t in range(1, N_DEV):
                pl.semaphore_signal(
                    barrier_sem, inc=1,
                    device_id=((my_i + t) % N_DEV,),
                    device_id_type=pl.DeviceIdType.MESH,
                )
            pl.semaphore_wait(barrier_sem, N_DEV - 1)

        q_all = lax.dot_general(
            x_ref[...], wq_ref[...], (((1,), (0,)), ((), ())),
            preferred_element_type=jnp.float32,
        ).astype(jnp.bfloat16)

        for b in range(B):
            for h in range(H_LOC):
                g = h // G_SIZE
                q = q_all[b * SQ:(b + 1) * SQ, h * DH:(h + 1) * DH]
                k = k_ref[b, :, g, :]
                v = v_ref[b, :, g, :]
                s = lax.dot_general(
                    q, k, (((1,), (1,)), ((), ())),
                    preferred_element_type=jnp.float32,
                ) * SCALE
                m = jnp.max(s, axis=1, keepdims=True)
                p = jnp.exp(s - m)
                l = jnp.sum(p, axis=1, keepdims=True)
                o = lax.dot_general(
                    p.astype(jnp.bfloat16), v, (((1,), (0,)), ((), ())),
                    preferred_element_type=jnp.float32,
                )
                attn_ref[b * SQ:(b + 1) * SQ, h * DH:(h + 1) * DH] = (
                    o / l
                ).astype(jnp.bfloat16)

        part_ref[...] = lax.dot_general(
            attn_ref[...], wo_ref[...], (((1,), (0,)), ((), ())),
            preferred_element_type=jnp.float32,
        )

        if _STAGE < 2:
            out_ref[...] = part_ref[...]
            return

        for t in range(1, N_DEV):
            tgt = (my_i + t) % N_DEV
            pltpu.make_async_remote_copy(
                src_ref=part_ref.at[pl.ds(tgt * CHUNK, CHUNK), :],
                dst_ref=recv_ref.at[t],
                send_sem=send_a.at[t],
                recv_sem=recv_a.at[t],
                device_id=(tgt,),
                device_id_type=pl.DeviceIdType.MESH,
            ).start()

        acc = part_ref[pl.ds(my_i * CHUNK, CHUNK), :]
        for t in range(1, N_DEV):
            pltpu.make_async_remote_copy(
                src_ref=part_ref.at[pl.ds(0, CHUNK), :],
                dst_ref=recv_ref.at[t],
                send_sem=send_a.at[t],
                recv_sem=recv_a.at[t],
                device_id=(0,),
                device_id_type=pl.DeviceIdType.MESH,
            ).wait_recv()
            acc = acc + recv_ref[t]

        out_ref[pl.ds(my_i * CHUNK, CHUNK), :] = acc

        if _STAGE >= 3:
            for t in range(1, N_DEV):
                tgt = (my_i + t) % N_DEV
                pltpu.make_async_remote_copy(
                    src_ref=out_ref.at[pl.ds(my_i * CHUNK, CHUNK), :],
                    dst_ref=out_ref.at[pl.ds(my_i * CHUNK, CHUNK), :],
                    send_sem=send_b.at[t],
                    recv_sem=recv_b.at[t],
                    device_id=(tgt,),
                    device_id_type=pl.DeviceIdType.MESH,
                ).start()

            for t in range(1, N_DEV):
                src_id = (my_i - t) % N_DEV
                pltpu.make_async_remote_copy(
                    src_ref=out_ref.at[pl.ds(src_id * CHUNK, CHUNK), :],
                    dst_ref=out_ref.at[pl.ds(src_id * CHUNK, CHUNK), :],
                    send_sem=send_b.at[t],
                    recv_sem=recv_b.at[t],
                    device_id=(src_id,),
                    device_id_type=pl.DeviceIdType.MESH,
                ).wait_recv()

        for t in range(1, N_DEV):
            pltpu.make_async_remote_copy(
                src_ref=part_ref.at[pl.ds(0, CHUNK), :],
                dst_ref=recv_ref.at[t],
                send_sem=send_a.at[t],
                recv_sem=recv_a.at[t],
                device_id=(0,),
                device_id_type=pl.DeviceIdType.MESH,
            ).wait_send()
            if _STAGE >= 3:
                pltpu.make_async_remote_copy(
                    src_ref=out_ref.at[pl.ds(0, CHUNK), :],
                    dst_ref=out_ref.at[pl.ds(0, CHUNK), :],
                    send_sem=send_b.at[t],
                    recv_sem=recv_b.at[t],
                    device_id=(0,),
                    device_id_type=pl.DeviceIdType.MESH,
                ).wait_send()

    out = pl.pallas_call(
        body,
        out_shape=jax.ShapeDtypeStruct((ROWS, D), jnp.float32),
        in_specs=[pl.BlockSpec(memory_space=pltpu.VMEM)] * 5,
        out_specs=pl.BlockSpec(memory_space=pltpu.VMEM),
        scratch_shapes=[
            pltpu.VMEM((ROWS, H_LOC * DH), jnp.bfloat16),
            pltpu.VMEM((ROWS, D), jnp.float32),
            pltpu.VMEM((N_DEV, CHUNK, D), jnp.float32),
            pltpu.SemaphoreType.DMA((N_DEV,)),
            pltpu.SemaphoreType.DMA((N_DEV,)),
            pltpu.SemaphoreType.DMA((N_DEV,)),
            pltpu.SemaphoreType.DMA((N_DEV,)),
        ],
        compiler_params=pltpu.CompilerParams(
            collective_id=0,
            vmem_limit_bytes=60 * 1024 * 1024,
        ),
    )(x2, Wq_b, Wo_b, K_loc, V_loc)
    return out.reshape(B, SQ, D)


# device time: 134923 ns/iter; 1.5195x vs baseline; 1.5195x over previous
import os

import jax
import jax.numpy as jnp
from jax import lax
from jax.experimental import pallas as pl
from jax.experimental.pallas import tpu as pltpu

_STAGE = int(os.environ.get("KERNEL_STAGE", "3"))

N_DEV = 16
B, SQ, D = 4, 256, 1024
H_LOC = 8
G_SIZE = 4
HKV_LOC = 2
DH = 128
ROWS = B * SQ
CHUNK = ROWS // N_DEV
SCALE = 0.08838834764831843

MASKS = [1, 3, 4, 8]
BITS = [3, 2, 1, 0]
RS_ROWS = [512, 256, 128, 64]
RS_OFFS = [0, 512, 768, 896]
AG_ROWS = [64, 128, 256, 512]
AG_OFFS = [0, 64, 192, 448]


def kernel(x, Wq, Wo, K_ext, V_ext):
    my = lax.axis_index("i")
    x2 = x.reshape(ROWS, D).astype(jnp.bfloat16)
    Wq_b = Wq.astype(jnp.bfloat16)
    Wo_b = Wo.astype(jnp.bfloat16)
    K_loc = lax.dynamic_slice_in_dim(K_ext, my * HKV_LOC, HKV_LOC, axis=2)
    V_loc = lax.dynamic_slice_in_dim(V_ext, my * HKV_LOC, HKV_LOC, axis=2)
    K_loc = K_loc.transpose(0, 2, 1, 3).astype(jnp.bfloat16)
    V_loc = V_loc.transpose(0, 2, 1, 3).astype(jnp.bfloat16)

    def body(x_ref, wq_ref, wo_ref, k_ref, v_ref, out_ref,
             q_ref, attn_ref, part_ref, send_stage, rs_recv, ag_recv,
             send_sems, recv_sems):
        my_i = lax.axis_index("i")

        if _STAGE >= 1:
            barrier_sem = pltpu.get_barrier_semaphore()
            for m in MASKS:
                pl.semaphore_signal(
                    barrier_sem, inc=1,
                    device_id=(my_i ^ m,),
                    device_id_type=pl.DeviceIdType.MESH,
                )
            pl.semaphore_wait(barrier_sem, len(MASKS))

        q_ref[...] = lax.dot_general(
            x_ref[...], wq_ref[...], (((1,), (0,)), ((), ())),
            preferred_element_type=jnp.float32,
        ).astype(jnp.bfloat16)

        def head_body(idx, carry):
            b = idx // H_LOC
            h = idx % H_LOC
            g = h // G_SIZE
            q = q_ref[pl.ds(b * SQ, SQ), pl.ds(h * DH, DH)]
            k = k_ref[b, g, :, :]
            v = v_ref[b, g, :, :]
            s = lax.dot_general(
                q, k, (((1,), (1,)), ((), ())),
                preferred_element_type=jnp.float32,
            ) * SCALE
            m = jnp.max(s, axis=1, keepdims=True)
            p = jnp.exp(s - m)
            l = jnp.sum(p, axis=1, keepdims=True)
            o = lax.dot_general(
                p.astype(jnp.bfloat16), v, (((1,), (0,)), ((), ())),
                preferred_element_type=jnp.float32,
            )
            attn_ref[pl.ds(b * SQ, SQ), pl.ds(h * DH, DH)] = (
                o / l
            ).astype(jnp.bfloat16)
            return carry

        lax.fori_loop(0, B * H_LOC, head_body, 0)

        part_ref[...] = lax.dot_general(
            attn_ref[...], wo_ref[...], (((1,), (0,)), ((), ())),
            preferred_element_type=jnp.float32,
        )

        if _STAGE < 2:
            out_ref[...] = part_ref[...]
            return

        i0 = my_i & 1
        i1 = (my_i >> 1) & 1
        i2 = (my_i >> 2) & 1
        i3 = (my_i >> 3) & 1
        v = (i0 ^ i1) * 8 + i1 * 4 + i2 * 2 + i3

        cur_lo = jnp.int32(0)
        for step in range(4):
            rows = RS_ROWS[step]
            off = RS_OFFS[step]
            half = rows // CHUNK
            my_bit = (v >> BITS[step]) & 1
            keep_lo = cur_lo + my_bit * half
            send_lo = cur_lo + (1 - my_bit) * half
            partner = my_i ^ MASKS[step]

            send_stage[off:off + rows, :] = part_ref[
                pl.ds(send_lo * CHUNK, rows), :
            ].astype(jnp.bfloat16)
            rdma = pltpu.make_async_remote_copy(
                src_ref=send_stage.at[pl.ds(off, rows), :],
                dst_ref=rs_recv.at[pl.ds(off, rows), :],
                send_sem=send_sems.at[step],
                recv_sem=recv_sems.at[step],
                device_id=(partner,),
                device_id_type=pl.DeviceIdType.MESH,
            )
            rdma.start()
            rdma.wait_recv()
            part_ref[pl.ds(keep_lo * CHUNK, rows), :] = (
                part_ref[pl.ds(keep_lo * CHUNK, rows), :]
                + rs_recv[off:off + rows, :].astype(jnp.float32)
            )
            cur_lo = keep_lo

        out_ref[pl.ds(v * CHUNK, CHUNK), :] = part_ref[
            pl.ds(cur_lo * CHUNK, CHUNK), :
        ]

        if _STAGE < 3:
            return

        for step in range(4):
            pltpu.make_async_remote_copy(
                src_ref=send_stage.at[pl.ds(RS_OFFS[step], RS_ROWS[step]), :],
                dst_ref=rs_recv.at[pl.ds(RS_OFFS[step], RS_ROWS[step]), :],
                send_sem=send_sems.at[step],
                recv_sem=recv_sems.at[step],
                device_id=(my_i,),
                device_id_type=pl.DeviceIdType.MESH,
            ).wait_send()

        lo = v
        for step in range(4):
            rows = AG_ROWS[step]
            off = AG_OFFS[step]
            mask = MASKS[3 - step]
            partner = my_i ^ mask

            send_stage[off:off + rows, :] = out_ref[
                pl.ds(lo * CHUNK, rows), :
            ].astype(jnp.bfloat16)
            rdma = pltpu.make_async_remote_copy(
                src_ref=send_stage.at[pl.ds(off, rows), :],
                dst_ref=ag_recv.at[pl.ds(off, rows), :],
                send_sem=send_sems.at[4 + step],
                recv_sem=recv_sems.at[4 + step],
                device_id=(partner,),
                device_id_type=pl.DeviceIdType.MESH,
            )
            rdma.start()
            rdma.wait_recv()
            plo = lo ^ (1 << step)
            out_ref[pl.ds(plo * CHUNK, rows), :] = ag_recv[
                off:off + rows, :
            ].astype(jnp.float32)
            lo = jnp.minimum(lo, plo)

        for step in range(4):
            pltpu.make_async_remote_copy(
                src_ref=send_stage.at[pl.ds(AG_OFFS[step], AG_ROWS[step]), :],
                dst_ref=ag_recv.at[pl.ds(AG_OFFS[step], AG_ROWS[step]), :],
                send_sem=send_sems.at[4 + step],
                recv_sem=recv_sems.at[4 + step],
                device_id=(my_i,),
                device_id_type=pl.DeviceIdType.MESH,
            ).wait_send()

    out = pl.pallas_call(
        body,
        out_shape=jax.ShapeDtypeStruct((ROWS, D), jnp.float32),
        in_specs=[pl.BlockSpec(memory_space=pltpu.VMEM)] * 5,
        out_specs=pl.BlockSpec(memory_space=pltpu.VMEM),
        scratch_shapes=[
            pltpu.VMEM((ROWS, H_LOC * DH), jnp.bfloat16),
            pltpu.VMEM((ROWS, H_LOC * DH), jnp.bfloat16),
            pltpu.VMEM((ROWS, D), jnp.float32),
            pltpu.VMEM((960, D), jnp.bfloat16),
            pltpu.VMEM((960, D), jnp.bfloat16),
            pltpu.VMEM((960, D), jnp.bfloat16),
            pltpu.SemaphoreType.DMA((8,)),
            pltpu.SemaphoreType.DMA((8,)),
        ],
        compiler_params=pltpu.CompilerParams(
            collective_id=0,
            vmem_limit_bytes=62 * 1024 * 1024,
        ),
    )(x2, Wq_b, Wo_b, K_loc, V_loc)
    return out.reshape(B, SQ, D)


# device time: 133804 ns/iter; 1.5322x vs baseline; 1.0084x over previous
import os

import jax
import jax.numpy as jnp
from jax import lax
from jax.experimental import pallas as pl
from jax.experimental.pallas import tpu as pltpu

_STAGE = int(os.environ.get("KERNEL_STAGE", "3"))

N_DEV = 16
B, SQ, D = 4, 256, 1024
H_LOC = 8
G_SIZE = 4
HKV_LOC = 2
DH = 128
ROWS = B * SQ
CHUNK = ROWS // N_DEV
SCALE = 0.08838834764831843

MASKS = [1, 3, 4, 8]
BITS = [3, 2, 1, 0]
RS_ROWS = [512, 256, 128, 64]
RS_OFFS = [0, 512, 768, 896]
AG_ROWS = [64, 128, 256, 512]
AG_OFFS = [0, 64, 192, 448]


def kernel(x, Wq, Wo, K_ext, V_ext):
    my = lax.axis_index("i")
    x2 = x.reshape(ROWS, D).astype(jnp.bfloat16)
    Wq_b = Wq.astype(jnp.bfloat16)
    Wo_b = Wo.astype(jnp.bfloat16)
    K_loc = lax.dynamic_slice_in_dim(K_ext, my * HKV_LOC, HKV_LOC, axis=2)
    V_loc = lax.dynamic_slice_in_dim(V_ext, my * HKV_LOC, HKV_LOC, axis=2)
    K_loc = K_loc.transpose(0, 2, 1, 3).astype(jnp.bfloat16)
    V_loc = V_loc.transpose(0, 2, 1, 3).astype(jnp.bfloat16)

    def body(x_ref, wq_ref, wo_ref, k_ref, v_ref, out_ref,
             q_ref, attn_ref, part_ref, send_stage, rs_recv, ag_recv,
             send_sems, recv_sems):
        my_i = lax.axis_index("i")

        if _STAGE >= 1:
            barrier_sem = pltpu.get_barrier_semaphore()
            for m in MASKS:
                pl.semaphore_signal(
                    barrier_sem, inc=1,
                    device_id=(my_i ^ m,),
                    device_id_type=pl.DeviceIdType.MESH,
                )
            pl.semaphore_wait(barrier_sem, len(MASKS))

        q_ref[...] = lax.dot_general(
            x_ref[...], wq_ref[...], (((1,), (0,)), ((), ())),
            preferred_element_type=jnp.float32,
        ).astype(jnp.bfloat16)

        GROUP_COLS = G_SIZE * DH
        GR = G_SIZE * SQ

        def group_body(idx, carry):
            b = idx // HKV_LOC
            g = idx % HKV_LOC
            q = q_ref[
                pl.ds(b * SQ, SQ), pl.ds(g * GROUP_COLS, GROUP_COLS)
            ].reshape(GR, DH)
            k = k_ref[b, g, :, :]
            v = v_ref[b, g, :, :]
            s = lax.dot_general(
                q, k, (((1,), (1,)), ((), ())),
                preferred_element_type=jnp.float32,
            ) * SCALE
            m = jnp.max(s, axis=1, keepdims=True)
            p = jnp.exp(s - m)
            l = jnp.sum(p, axis=1, keepdims=True)
            o = lax.dot_general(
                p.astype(jnp.bfloat16), v, (((1,), (0,)), ((), ())),
                preferred_element_type=jnp.float32,
            )
            attn_ref[pl.ds(b * SQ, SQ), pl.ds(g * GROUP_COLS, GROUP_COLS)] = (
                (o / l).astype(jnp.bfloat16).reshape(SQ, GROUP_COLS)
            )
            return carry

        lax.fori_loop(0, B * HKV_LOC, group_body, 0)

        part_ref[...] = lax.dot_general(
            attn_ref[...], wo_ref[...], (((1,), (0,)), ((), ())),
            preferred_element_type=jnp.float32,
        )

        if _STAGE < 2:
            out_ref[...] = part_ref[...]
            return

        i0 = my_i & 1
        i1 = (my_i >> 1) & 1
        i2 = (my_i >> 2) & 1
        i3 = (my_i >> 3) & 1
        v = (i0 ^ i1) * 8 + i1 * 4 + i2 * 2 + i3

        cur_lo = jnp.int32(0)
        for step in range(4):
            rows = RS_ROWS[step]
            off = RS_OFFS[step]
            half = rows // CHUNK
            my_bit = (v >> BITS[step]) & 1
            keep_lo = cur_lo + my_bit * half
            send_lo = cur_lo + (1 - my_bit) * half
            partner = my_i ^ MASKS[step]

            send_stage[off:off + rows, :] = part_ref[
                pl.ds(send_lo * CHUNK, rows), :
            ].astype(jnp.bfloat16)
            rdma = pltpu.make_async_remote_copy(
                src_ref=send_stage.at[pl.ds(off, rows), :],
                dst_ref=rs_recv.at[pl.ds(off, rows), :],
                send_sem=send_sems.at[step],
                recv_sem=recv_sems.at[step],
                device_id=(partner,),
                device_id_type=pl.DeviceIdType.MESH,
            )
            rdma.start()
            rdma.wait_recv()
            part_ref[pl.ds(keep_lo * CHUNK, rows), :] = (
                part_ref[pl.ds(keep_lo * CHUNK, rows), :]
                + rs_recv[off:off + rows, :].astype(jnp.float32)
            )
            cur_lo = keep_lo

        out_ref[pl.ds(v * CHUNK, CHUNK), :] = part_ref[
            pl.ds(cur_lo * CHUNK, CHUNK), :
        ]

        if _STAGE < 3:
            return

        for step in range(4):
            pltpu.make_async_remote_copy(
                src_ref=send_stage.at[pl.ds(RS_OFFS[step], RS_ROWS[step]), :],
                dst_ref=rs_recv.at[pl.ds(RS_OFFS[step], RS_ROWS[step]), :],
                send_sem=send_sems.at[step],
                recv_sem=recv_sems.at[step],
                device_id=(my_i,),
                device_id_type=pl.DeviceIdType.MESH,
            ).wait_send()

        lo = v
        for step in range(4):
            rows = AG_ROWS[step]
            off = AG_OFFS[step]
            mask = MASKS[3 - step]
            partner = my_i ^ mask

            send_stage[off:off + rows, :] = out_ref[
                pl.ds(lo * CHUNK, rows), :
            ].astype(jnp.bfloat16)
            rdma = pltpu.make_async_remote_copy(
                src_ref=send_stage.at[pl.ds(off, rows), :],
                dst_ref=ag_recv.at[pl.ds(off, rows), :],
                send_sem=send_sems.at[4 + step],
                recv_sem=recv_sems.at[4 + step],
                device_id=(partner,),
                device_id_type=pl.DeviceIdType.MESH,
            )
            rdma.start()
            rdma.wait_recv()
            plo = lo ^ (1 << step)
            out_ref[pl.ds(plo * CHUNK, rows), :] = ag_recv[
                off:off + rows, :
            ].astype(jnp.float32)
            lo = jnp.minimum(lo, plo)

        for step in range(4):
            pltpu.make_async_remote_copy(
                src_ref=send_stage.at[pl.ds(AG_OFFS[step], AG_ROWS[step]), :],
                dst_ref=ag_recv.at[pl.ds(AG_OFFS[step], AG_ROWS[step]), :],
                send_sem=send_sems.at[4 + step],
                recv_sem=recv_sems.at[4 + step],
                device_id=(my_i,),
                device_id_type=pl.DeviceIdType.MESH,
            ).wait_send()

    out = pl.pallas_call(
        body,
        out_shape=jax.ShapeDtypeStruct((ROWS, D), jnp.float32),
        in_specs=[pl.BlockSpec(memory_space=pltpu.VMEM)] * 5,
        out_specs=pl.BlockSpec(memory_space=pltpu.VMEM),
        scratch_shapes=[
            pltpu.VMEM((ROWS, H_LOC * DH), jnp.bfloat16),
            pltpu.VMEM((ROWS, H_LOC * DH), jnp.bfloat16),
            pltpu.VMEM((ROWS, D), jnp.float32),
            pltpu.VMEM((960, D), jnp.bfloat16),
            pltpu.VMEM((960, D), jnp.bfloat16),
            pltpu.VMEM((960, D), jnp.bfloat16),
            pltpu.SemaphoreType.DMA((8,)),
            pltpu.SemaphoreType.DMA((8,)),
        ],
        compiler_params=pltpu.CompilerParams(
            collective_id=0,
            vmem_limit_bytes=62 * 1024 * 1024,
        ),
    )(x2, Wq_b, Wo_b, K_loc, V_loc)
    return out.reshape(B, SQ, D)


# device time: 99292 ns/iter; 2.0648x vs baseline; 1.3476x over previous
import os

import jax
import jax.numpy as jnp
from jax import lax
from jax.experimental import pallas as pl
from jax.experimental.pallas import tpu as pltpu

_STAGE = int(os.environ.get("KERNEL_STAGE", "3"))

N_DEV = 16
B, SQ, D = 4, 256, 1024
H_LOC = 8
G_SIZE = 4
HKV_LOC = 2
DH = 128
ROWS = B * SQ
CHUNK = ROWS // N_DEV
SCALE = 0.08838834764831843

MASKS = [1, 3, 4, 8]
BITS = [3, 2, 1, 0]
RS_ROWS = [512, 256, 128, 64]
RS_OFFS = [0, 512, 768, 896]
AG_ROWS = [64, 128, 256, 512]
AG_OFFS = [0, 64, 192, 448]


def kernel(x, Wq, Wo, K_ext, V_ext):
    x2 = x.reshape(ROWS, D)

    def body(x_ref, wq_ref, wo_ref, khbm, vhbm, out_ref,
             q_ref, attn_ref, part_ref, send_stage, rs_recv, ag_recv,
             k_stage, v_stage, send_sems, recv_sems, kv_sems):
        my_i = lax.axis_index("i")

        kv_copies = []
        for g in range(HKV_LOC):
            h_idx = my_i * HKV_LOC + g
            ck = pltpu.make_async_copy(
                khbm.at[:, :, h_idx, :], k_stage.at[g], kv_sems.at[g]
            )
            cv = pltpu.make_async_copy(
                vhbm.at[:, :, h_idx, :], v_stage.at[g],
                kv_sems.at[HKV_LOC + g],
            )
            ck.start()
            cv.start()
            kv_copies += [ck, cv]

        if _STAGE >= 1:
            barrier_sem = pltpu.get_barrier_semaphore()
            for m in MASKS:
                pl.semaphore_signal(
                    barrier_sem, inc=1,
                    device_id=(my_i ^ m,),
                    device_id_type=pl.DeviceIdType.MESH,
                )
            pl.semaphore_wait(barrier_sem, len(MASKS))

        q_ref[...] = (
            lax.dot_general(
                x_ref[...].astype(jnp.bfloat16),
                wq_ref[...].astype(jnp.bfloat16),
                (((1,), (0,)), ((), ())),
                preferred_element_type=jnp.float32,
            )
            * SCALE
        ).astype(jnp.bfloat16)

        for c in kv_copies:
            c.wait()

        GROUP_COLS = G_SIZE * DH
        GR = G_SIZE * SQ

        def group_body(idx, carry):
            b = idx // HKV_LOC
            g = idx % HKV_LOC
            q = q_ref[
                pl.ds(b * SQ, SQ), pl.ds(g * GROUP_COLS, GROUP_COLS)
            ].reshape(GR, DH)
            k = k_stage[g, b, :, :].astype(jnp.bfloat16)
            v = v_stage[g, b, :, :].astype(jnp.bfloat16)
            s = lax.dot_general(
                q, k, (((1,), (1,)), ((), ())),
                preferred_element_type=jnp.float32,
            )
            m = jnp.max(s, axis=1, keepdims=True)
            p = jnp.exp(s - m)
            l = jnp.sum(p, axis=1, keepdims=True)
            o = lax.dot_general(
                p.astype(jnp.bfloat16), v, (((1,), (0,)), ((), ())),
                preferred_element_type=jnp.float32,
            )
            attn_ref[pl.ds(b * SQ, SQ), pl.ds(g * GROUP_COLS, GROUP_COLS)] = (
                (o / l).astype(jnp.bfloat16).reshape(SQ, GROUP_COLS)
            )
            return carry

        lax.fori_loop(0, B * HKV_LOC, group_body, 0)

        part_ref[...] = lax.dot_general(
            attn_ref[...], wo_ref[...].astype(jnp.bfloat16),
            (((1,), (0,)), ((), ())),
            preferred_element_type=jnp.float32,
        )

        if _STAGE < 2:
            out_ref[...] = part_ref[...]
            return

        i0 = my_i & 1
        i1 = (my_i >> 1) & 1
        i2 = (my_i >> 2) & 1
        i3 = (my_i >> 3) & 1
        v = (i0 ^ i1) * 8 + i1 * 4 + i2 * 2 + i3

        cur_lo = jnp.int32(0)
        for step in range(4):
            rows = RS_ROWS[step]
            off = RS_OFFS[step]
            half = rows // CHUNK
            my_bit = (v >> BITS[step]) & 1
            keep_lo = cur_lo + my_bit * half
            send_lo = cur_lo + (1 - my_bit) * half
            partner = my_i ^ MASKS[step]

            send_stage[off:off + rows, :] = part_ref[
                pl.ds(send_lo * CHUNK, rows), :
            ].astype(jnp.bfloat16)
            rdma = pltpu.make_async_remote_copy(
                src_ref=send_stage.at[pl.ds(off, rows), :],
                dst_ref=rs_recv.at[pl.ds(off, rows), :],
                send_sem=send_sems.at[step],
                recv_sem=recv_sems.at[step],
                device_id=(partner,),
                device_id_type=pl.DeviceIdType.MESH,
            )
            rdma.start()
            rdma.wait_recv()
            part_ref[pl.ds(keep_lo * CHUNK, rows), :] = (
                part_ref[pl.ds(keep_lo * CHUNK, rows), :]
                + rs_recv[off:off + rows, :].astype(jnp.float32)
            )
            cur_lo = keep_lo

        out_ref[pl.ds(v * CHUNK, CHUNK), :] = part_ref[
            pl.ds(cur_lo * CHUNK, CHUNK), :
        ]

        if _STAGE < 3:
            return

        for step in range(4):
            pltpu.make_async_remote_copy(
                src_ref=send_stage.at[pl.ds(RS_OFFS[step], RS_ROWS[step]), :],
                dst_ref=rs_recv.at[pl.ds(RS_OFFS[step], RS_ROWS[step]), :],
                send_sem=send_sems.at[step],
                recv_sem=recv_sems.at[step],
                device_id=(my_i,),
                device_id_type=pl.DeviceIdType.MESH,
            ).wait_send()

        lo = v
        for step in range(4):
            rows = AG_ROWS[step]
            off = AG_OFFS[step]
            mask = MASKS[3 - step]
            partner = my_i ^ mask

            send_stage[off:off + rows, :] = out_ref[
                pl.ds(lo * CHUNK, rows), :
            ].astype(jnp.bfloat16)
            rdma = pltpu.make_async_remote_copy(
                src_ref=send_stage.at[pl.ds(off, rows), :],
                dst_ref=ag_recv.at[pl.ds(off, rows), :],
                send_sem=send_sems.at[4 + step],
                recv_sem=recv_sems.at[4 + step],
                device_id=(partner,),
                device_id_type=pl.DeviceIdType.MESH,
            )
            rdma.start()
            rdma.wait_recv()
            plo = lo ^ (1 << step)
            out_ref[pl.ds(plo * CHUNK, rows), :] = ag_recv[
                off:off + rows, :
            ].astype(jnp.float32)
            lo = jnp.minimum(lo, plo)

        for step in range(4):
            pltpu.make_async_remote_copy(
                src_ref=send_stage.at[pl.ds(AG_OFFS[step], AG_ROWS[step]), :],
                dst_ref=ag_recv.at[pl.ds(AG_OFFS[step], AG_ROWS[step]), :],
                send_sem=send_sems.at[4 + step],
                recv_sem=recv_sems.at[4 + step],
                device_id=(my_i,),
                device_id_type=pl.DeviceIdType.MESH,
            ).wait_send()

    out = pl.pallas_call(
        body,
        out_shape=jax.ShapeDtypeStruct((ROWS, D), jnp.float32),
        in_specs=[
            pl.BlockSpec(memory_space=pltpu.VMEM),
            pl.BlockSpec(memory_space=pltpu.VMEM),
            pl.BlockSpec(memory_space=pltpu.VMEM),
            pl.BlockSpec(memory_space=pl.ANY),
            pl.BlockSpec(memory_space=pl.ANY),
        ],
        out_specs=pl.BlockSpec(memory_space=pltpu.VMEM),
        scratch_shapes=[
            pltpu.VMEM((ROWS, H_LOC * DH), jnp.bfloat16),
            pltpu.VMEM((ROWS, H_LOC * DH), jnp.bfloat16),
            pltpu.VMEM((ROWS, D), jnp.float32),
            pltpu.VMEM((960, D), jnp.bfloat16),
            pltpu.VMEM((960, D), jnp.bfloat16),
            pltpu.VMEM((960, D), jnp.bfloat16),
            pltpu.VMEM((HKV_LOC, B, 1024, DH), jnp.float32),
            pltpu.VMEM((HKV_LOC, B, 1024, DH), jnp.float32),
            pltpu.SemaphoreType.DMA((8,)),
            pltpu.SemaphoreType.DMA((8,)),
            pltpu.SemaphoreType.DMA((2 * HKV_LOC,)),
        ],
        compiler_params=pltpu.CompilerParams(
            collective_id=0 if _STAGE >= 1 else None,
            vmem_limit_bytes=62 * 1024 * 1024,
        ),
    )(x2, Wq, Wo, K_ext, V_ext)
    return out.reshape(B, SQ, D)


# device time: 87665 ns/iter; 2.3386x vs baseline; 1.1326x over previous
import os

import jax
import jax.numpy as jnp
from jax import lax
from jax.experimental import pallas as pl
from jax.experimental.pallas import tpu as pltpu

_STAGE = int(os.environ.get("KERNEL_STAGE", "3"))

N_DEV = 16
B, SQ, D = 4, 256, 1024
H_LOC = 8
G_SIZE = 4
HKV_LOC = 2
DH = 128
ROWS = B * SQ
CHUNK = ROWS // N_DEV
SCALE = 0.08838834764831843

MASKS = [1, 3, 4, 8]
BITS = [3, 2, 1, 0]
RS_ROWS = [512, 256, 128, 64]
RS_OFFS = [0, 512, 768, 896]
AG_ROWS = [64, 128, 256, 512]
AG_OFFS = [0, 64, 192, 448]


def kernel(x, Wq, Wo, K_ext, V_ext):
    x2 = x.reshape(ROWS, D)

    def body(x_ref, wq_ref, wo_ref, khbm, vhbm, out_ref,
             q_ref, attn_ref, part_ref, send_stage, rs_recv, ag_recv,
             k_stage, v_stage, send_sems, recv_sems, kv_sems):
        my_i = lax.axis_index("i")

        kv_copies = []
        for g in range(HKV_LOC):
            h_idx = my_i * HKV_LOC + g
            ck = pltpu.make_async_copy(
                khbm.at[:, :, h_idx, :], k_stage.at[g], kv_sems.at[g]
            )
            cv = pltpu.make_async_copy(
                vhbm.at[:, :, h_idx, :], v_stage.at[g],
                kv_sems.at[HKV_LOC + g],
            )
            ck.start()
            cv.start()
            kv_copies += [ck, cv]

        if _STAGE >= 1:
            barrier_sem = pltpu.get_barrier_semaphore()
            for m in MASKS:
                pl.semaphore_signal(
                    barrier_sem, inc=1,
                    device_id=(my_i ^ m,),
                    device_id_type=pl.DeviceIdType.MESH,
                )
            pl.semaphore_wait(barrier_sem, len(MASKS))

        q_ref[...] = (
            lax.dot_general(
                x_ref[...].astype(jnp.bfloat16),
                wq_ref[...].astype(jnp.bfloat16),
                (((1,), (0,)), ((), ())),
                preferred_element_type=jnp.float32,
            )
            * SCALE
        ).astype(jnp.bfloat16)

        for c in kv_copies:
            c.wait()

        i0 = my_i & 1
        i1 = (my_i >> 1) & 1
        i2 = (my_i >> 2) & 1
        i3 = (my_i >> 3) & 1
        v = (i0 ^ i1) * 8 + i1 * 4 + i2 * 2 + i3

        my_bit0 = (v >> BITS[0]) & 1
        keep_lo0 = my_bit0 * 8
        send_lo0 = (1 - my_bit0) * 8
        bs = (1 - my_bit0) * 2
        bk = my_bit0 * 2

        GROUP_COLS = G_SIZE * DH
        GR = G_SIZE * SQ

        def make_group_body(b_base):
            def group_body(idx, carry):
                b = b_base + idx // HKV_LOC
                g = idx % HKV_LOC
                q = q_ref[
                    pl.ds(b * SQ, SQ), pl.ds(g * GROUP_COLS, GROUP_COLS)
                ].reshape(GR, DH)
                k = k_stage[g, b, :, :].astype(jnp.bfloat16)
                v_ = v_stage[g, b, :, :].astype(jnp.bfloat16)
                s = lax.dot_general(
                    q, k, (((1,), (1,)), ((), ())),
                    preferred_element_type=jnp.float32,
                )
                m = jnp.max(s, axis=1, keepdims=True)
                p = jnp.exp(s - m)
                l = jnp.sum(p, axis=1, keepdims=True)
                o = lax.dot_general(
                    p.astype(jnp.bfloat16), v_, (((1,), (0,)), ((), ())),
                    preferred_element_type=jnp.float32,
                )
                attn_ref[
                    pl.ds(b * SQ, SQ), pl.ds(g * GROUP_COLS, GROUP_COLS)
                ] = (o / l).astype(jnp.bfloat16).reshape(SQ, GROUP_COLS)
                return carry

            return group_body

        def proj_half(b0):
            part_ref[pl.ds(b0 * SQ, 2 * SQ), :] = lax.dot_general(
                attn_ref[pl.ds(b0 * SQ, 2 * SQ), :],
                wo_ref[...].astype(jnp.bfloat16),
                (((1,), (0,)), ((), ())),
                preferred_element_type=jnp.float32,
            )

        lax.fori_loop(0, 2 * HKV_LOC, make_group_body(bs), 0)
        proj_half(bs)

        if _STAGE >= 2:
            send_stage[0:RS_ROWS[0], :] = part_ref[
                pl.ds(send_lo0 * CHUNK, RS_ROWS[0]), :
            ].astype(jnp.bfloat16)
            rdma0 = pltpu.make_async_remote_copy(
                src_ref=send_stage.at[pl.ds(0, RS_ROWS[0]), :],
                dst_ref=rs_recv.at[pl.ds(0, RS_ROWS[0]), :],
                send_sem=send_sems.at[0],
                recv_sem=recv_sems.at[0],
                device_id=(my_i ^ MASKS[0],),
                device_id_type=pl.DeviceIdType.MESH,
            )
            rdma0.start()

        lax.fori_loop(0, 2 * HKV_LOC, make_group_body(bk), 0)
        proj_half(bk)

        if _STAGE < 2:
            out_ref[...] = part_ref[...]
            return

        rdma0.wait_recv()
        part_ref[pl.ds(keep_lo0 * CHUNK, RS_ROWS[0]), :] = (
            part_ref[pl.ds(keep_lo0 * CHUNK, RS_ROWS[0]), :]
            + rs_recv[0:RS_ROWS[0], :].astype(jnp.float32)
        )
        cur_lo = keep_lo0

        for step in range(1, 4):
            rows = RS_ROWS[step]
            off = RS_OFFS[step]
            half = rows // CHUNK
            my_bit = (v >> BITS[step]) & 1
            keep_lo = cur_lo + my_bit * half
            send_lo = cur_lo + (1 - my_bit) * half
            partner = my_i ^ MASKS[step]

            send_stage[off:off + rows, :] = part_ref[
                pl.ds(send_lo * CHUNK, rows), :
            ].astype(jnp.bfloat16)
            rdma = pltpu.make_async_remote_copy(
                src_ref=send_stage.at[pl.ds(off, rows), :],
                dst_ref=rs_recv.at[pl.ds(off, rows), :],
                send_sem=send_sems.at[step],
                recv_sem=recv_sems.at[step],
                device_id=(partner,),
                device_id_type=pl.DeviceIdType.MESH,
            )
            rdma.start()
            rdma.wait_recv()
            part_ref[pl.ds(keep_lo * CHUNK, rows), :] = (
                part_ref[pl.ds(keep_lo * CHUNK, rows), :]
                + rs_recv[off:off + rows, :].astype(jnp.float32)
            )
            cur_lo = keep_lo

        out_ref[pl.ds(v * CHUNK, CHUNK), :] = part_ref[
            pl.ds(cur_lo * CHUNK, CHUNK), :
        ]

        if _STAGE < 3:
            return

        for step in range(4):
            pltpu.make_async_remote_copy(
                src_ref=send_stage.at[pl.ds(RS_OFFS[step], RS_ROWS[step]), :],
                dst_ref=rs_recv.at[pl.ds(RS_OFFS[step], RS_ROWS[step]), :],
                send_sem=send_sems.at[step],
                recv_sem=recv_sems.at[step],
                device_id=(my_i,),
                device_id_type=pl.DeviceIdType.MESH,
            ).wait_send()

        lo = v
        for step in range(4):
            rows = AG_ROWS[step]
            off = AG_OFFS[step]
            mask = MASKS[3 - step]
            partner = my_i ^ mask

            send_stage[off:off + rows, :] = out_ref[
                pl.ds(lo * CHUNK, rows), :
            ].astype(jnp.bfloat16)
            rdma = pltpu.make_async_remote_copy(
                src_ref=send_stage.at[pl.ds(off, rows), :],
                dst_ref=ag_recv.at[pl.ds(off, rows), :],
                send_sem=send_sems.at[4 + step],
                recv_sem=recv_sems.at[4 + step],
                device_id=(partner,),
                device_id_type=pl.DeviceIdType.MESH,
            )
            rdma.start()
            rdma.wait_recv()
            plo = lo ^ (1 << step)
            out_ref[pl.ds(plo * CHUNK, rows), :] = ag_recv[
                off:off + rows, :
            ].astype(jnp.float32)
            lo = jnp.minimum(lo, plo)

        for step in range(4):
            pltpu.make_async_remote_copy(
                src_ref=send_stage.at[pl.ds(AG_OFFS[step], AG_ROWS[step]), :],
                dst_ref=ag_recv.at[pl.ds(AG_OFFS[step], AG_ROWS[step]), :],
                send_sem=send_sems.at[4 + step],
                recv_sem=recv_sems.at[4 + step],
                device_id=(my_i,),
                device_id_type=pl.DeviceIdType.MESH,
            ).wait_send()

    out = pl.pallas_call(
        body,
        out_shape=jax.ShapeDtypeStruct((ROWS, D), jnp.float32),
        in_specs=[
            pl.BlockSpec(memory_space=pltpu.VMEM),
            pl.BlockSpec(memory_space=pltpu.VMEM),
            pl.BlockSpec(memory_space=pltpu.VMEM),
            pl.BlockSpec(memory_space=pl.ANY),
            pl.BlockSpec(memory_space=pl.ANY),
        ],
        out_specs=pl.BlockSpec(memory_space=pltpu.VMEM),
        scratch_shapes=[
            pltpu.VMEM((ROWS, H_LOC * DH), jnp.bfloat16),
            pltpu.VMEM((ROWS, H_LOC * DH), jnp.bfloat16),
            pltpu.VMEM((ROWS, D), jnp.float32),
            pltpu.VMEM((960, D), jnp.bfloat16),
            pltpu.VMEM((960, D), jnp.bfloat16),
            pltpu.VMEM((960, D), jnp.bfloat16),
            pltpu.VMEM((HKV_LOC, B, 1024, DH), jnp.float32),
            pltpu.VMEM((HKV_LOC, B, 1024, DH), jnp.float32),
            pltpu.SemaphoreType.DMA((8,)),
            pltpu.SemaphoreType.DMA((8,)),
            pltpu.SemaphoreType.DMA((2 * HKV_LOC,)),
        ],
        compiler_params=pltpu.CompilerParams(
            collective_id=0 if _STAGE >= 1 else None,
            vmem_limit_bytes=62 * 1024 * 1024,
        ),
    )(x2, Wq, Wo, K_ext, V_ext)
    return out.reshape(B, SQ, D)
